# baseline (speedup 1.0000x reference)
import jax
import jax.numpy as jnp
from jax.experimental import pallas as pl
from jax.experimental.pallas import tpu as pltpu
from jax.experimental.pallas import tpu_sc as plsc

EMBED_DIM = 64
RANK_DIM = 8
SCALING = 16.0 / 8.0
GATHER_WINDOW = 128
OUT_BATCH = 64


def _tc_m(a_t, b_t):
    def body(at_ref, bt_ref, m_ref):
        eye = (jax.lax.broadcasted_iota(jnp.int32, (EMBED_DIM, EMBED_DIM), 0)
               == jax.lax.broadcasted_iota(
                   jnp.int32, (EMBED_DIM, EMBED_DIM), 1)).astype(jnp.float32)
        m_ref[...] = eye + SCALING * jnp.dot(
            at_ref[...], bt_ref[...], preferred_element_type=jnp.float32)

    return pl.pallas_call(
        body,
        out_shape=jax.ShapeDtypeStruct((EMBED_DIM, EMBED_DIM), jnp.float32),
    )(a_t, b_t)


def _pack_bf16(table):
    quarter = table.shape[0] // 4
    u = jax.lax.bitcast_convert_type(table.astype(jnp.bfloat16), jnp.uint16)
    lo = u[:, :32].astype(jnp.uint32)
    hi = u[:, 32:].astype(jnp.uint32)
    w = jax.lax.bitcast_convert_type(lo | (hi << 16), jnp.float32)
    return jnp.concatenate(
        [w[k * quarter:(k + 1) * quarter] for k in range(4)], axis=1)


def _sc_gather(table_wide, idx_q):
    n = idx_q.shape[0]
    width = table_wide.shape[1]
    indices = idx_q.reshape(1, n)
    mesh = plsc.VectorSubcoreMesh(core_axis_name="core",
                                  subcore_axis_name="subcore")

    @pl.kernel(out_type=jax.ShapeDtypeStruct((n, width), table_wide.dtype),
               mesh=mesh)
    def gather_kernel(tab_hbm, i_hbm, o_hbm):
        def body(i_vmem, o_vmem):
            pltpu.sync_copy(tab_hbm.at[i_vmem.at[0]], o_vmem)

        pltpu.emit_pipeline(
            body,
            grid=(n // GATHER_WINDOW,),
            in_specs=[pl.BlockSpec((1, GATHER_WINDOW), lambda i: (0, i))],
            out_specs=[pl.BlockSpec((GATHER_WINDOW, width),
                                    lambda i: (i, 0))],
            core_axis_name=("core", "subcore"),
            dimension_semantics=(pltpu.PARALLEL,),
        )(i_hbm, o_hbm)

    return gather_kernel(table_wide, indices)


def _tc_select_lora(g_wide, b1_t, b0_t, m, bsz, seq):
    rows_per_block = OUT_BATCH * seq
    par_cols = rows_per_block // 128

    def body(g_ref, b1_ref, b0_ref, m_ref, o_ref):
        u = jax.lax.bitcast_convert_type(g_ref[...], jnp.uint32)
        lo_f = jax.lax.bitcast_convert_type(u << 16, jnp.float32)
        hi_f = jax.lax.bitcast_convert_type(
            u & jnp.uint32(0xFFFF0000), jnp.float32)
        b1 = jnp.concatenate(
            [b1_ref[0, :, j:j + 1] for j in range(par_cols)], axis=0)
        b0 = jnp.concatenate(
            [b0_ref[0, :, j:j + 1] for j in range(par_cols)], axis=0)
        n1, n0 = 1.0 - b1, 1.0 - b0
        q = (n1 * n0, n1 * b0, b1 * n0, b1 * b0)
        sel_lo = sum(q[k] * lo_f[:, 32 * k:32 * k + 32] for k in range(4))
        sel_hi = sum(q[k] * hi_f[:, 32 * k:32 * k + 32] for k in range(4))
        sel = jnp.concatenate([sel_lo, sel_hi], axis=1)
        out = jnp.dot(sel, m_ref[...], preferred_element_type=jnp.float32)
        o_ref[...] = out.reshape(OUT_BATCH, seq, EMBED_DIM)

    return pl.pallas_call(
        body,
        grid=(bsz // OUT_BATCH,),
        in_specs=[
            pl.BlockSpec((rows_per_block, 2 * EMBED_DIM), lambda i: (i, 0)),
            pl.BlockSpec((1, 128, par_cols), lambda i: (i, 0, 0)),
            pl.BlockSpec((1, 128, par_cols), lambda i: (i, 0, 0)),
            pl.BlockSpec((EMBED_DIM, EMBED_DIM), lambda i: (0, 0)),
        ],
        out_specs=pl.BlockSpec((OUT_BATCH, seq, EMBED_DIM),
                               lambda i: (i, 0, 0)),
        out_shape=jax.ShapeDtypeStruct((bsz, seq, EMBED_DIM), jnp.float32),
    )(g_wide, b1_t, b0_t, m)


def _bit_cols(bits_f32, n, bsz, par_cols):
    return (bits_f32
            .reshape(n // 128, 128).T
            .reshape(128, bsz // OUT_BATCH, par_cols)
            .transpose(1, 0, 2))


def kernel(x, table, lora_A, lora_B):
    bsz, seq = x.shape
    n = bsz * seq
    par_cols = OUT_BATCH * seq // 128
    quarter = table.shape[0] // 4
    idx = x.reshape(-1).astype(jnp.int32)
    qsel = idx // quarter
    b1_t = _bit_cols((qsel >> 1).astype(jnp.float32), n, bsz, par_cols)
    b0_t = _bit_cols((qsel & 1).astype(jnp.float32), n, bsz, par_cols)
    m = _tc_m(lora_A.T, lora_B.T)
    table_wide = _pack_bf16(table)
    g_wide = _sc_gather(table_wide, idx % quarter)
    return _tc_select_lora(g_wide, b1_t, b0_t, m, bsz, seq)


# speedup vs baseline: 2.1094x; 2.1094x over previous
import jax
import jax.numpy as jnp
from jax.experimental import pallas as pl
from jax.experimental.pallas import tpu as pltpu
from jax.experimental.pallas import tpu_sc as plsc

EMBED_DIM = 64
RANK_DIM = 8
SCALING = 16.0 / 8.0
GATHER_WINDOW = 128
OUT_BATCH = 64


def _tc_m_stack(a_t, b_t):
    def body(at_ref, bt_ref, m_ref):
        eye = (jax.lax.broadcasted_iota(jnp.int32, (EMBED_DIM, EMBED_DIM), 0)
               == jax.lax.broadcasted_iota(
                   jnp.int32, (EMBED_DIM, EMBED_DIM), 1)).astype(jnp.float32)
        m = eye + SCALING * jnp.dot(at_ref[...], bt_ref[...],
                                    preferred_element_type=jnp.float32)
        m_ref[...] = jnp.concatenate([m, m], axis=0)

    return pl.pallas_call(
        body,
        out_shape=jax.ShapeDtypeStruct((2 * EMBED_DIM, EMBED_DIM),
                                       jnp.float32),
    )(a_t, b_t)


def _sc_gather(table_wide, idx_half):
    n = idx_half.shape[0]
    width = table_wide.shape[1]
    indices = idx_half.reshape(1, n)
    mesh = plsc.VectorSubcoreMesh(core_axis_name="core",
                                  subcore_axis_name="subcore")

    @pl.kernel(out_type=jax.ShapeDtypeStruct((n, width), table_wide.dtype),
               mesh=mesh)
    def gather_kernel(tab_hbm, i_hbm, o_hbm):
        def body(i_vmem, o_vmem):
            pltpu.sync_copy(tab_hbm.at[i_vmem.at[0]], o_vmem)

        pltpu.emit_pipeline(
            body,
            grid=(n // GATHER_WINDOW,),
            in_specs=[pl.BlockSpec((1, GATHER_WINDOW), lambda i: (0, i))],
            out_specs=[pl.BlockSpec((GATHER_WINDOW, width),
                                    lambda i: (i, 0))],
            core_axis_name=("core", "subcore"),
            dimension_semantics=(pltpu.PARALLEL,),
        )(i_hbm, o_hbm)

    return gather_kernel(table_wide, indices)


def _tc_select_lora(g_wide, par_t, m_stack, bsz, seq):
    rows_per_block = OUT_BATCH * seq
    par_cols = rows_per_block // 128

    def body(g_ref, p_ref, m_ref, o_ref):
        gb = g_ref[...]
        parts = []
        for j in range(par_cols):
            lo, hi = j * 128, (j + 1) * 128
            p = p_ref[0, :, j:j + 1]
            mask = jnp.concatenate(
                [jnp.broadcast_to(1.0 - p, (128, EMBED_DIM)),
                 jnp.broadcast_to(p, (128, EMBED_DIM))], axis=1)
            parts.append(gb[lo:hi] * mask)
        sel = jnp.concatenate(parts, axis=0)
        out = jnp.dot(sel, m_ref[...], preferred_element_type=jnp.float32)
        o_ref[...] = out.reshape(OUT_BATCH, seq, EMBED_DIM)

    return pl.pallas_call(
        body,
        grid=(bsz // OUT_BATCH,),
        in_specs=[
            pl.BlockSpec((rows_per_block, 2 * EMBED_DIM), lambda i: (i, 0)),
            pl.BlockSpec((1, 128, par_cols), lambda i: (i, 0, 0)),
            pl.BlockSpec((2 * EMBED_DIM, EMBED_DIM), lambda i: (0, 0)),
        ],
        out_specs=pl.BlockSpec((OUT_BATCH, seq, EMBED_DIM),
                               lambda i: (i, 0, 0)),
        out_shape=jax.ShapeDtypeStruct((bsz, seq, EMBED_DIM), jnp.float32),
    )(g_wide, par_t, m_stack)


def kernel(x, table, lora_A, lora_B):
    bsz, seq = x.shape
    n = bsz * seq
    par_cols = OUT_BATCH * seq // 128
    idx = x.reshape(-1).astype(jnp.int32)
    par_t = ((idx & 1).astype(jnp.float32)
             .reshape(n // 128, 128).T
             .reshape(128, bsz // OUT_BATCH, par_cols)
             .transpose(1, 0, 2))
    m_stack = _tc_m_stack(lora_A.T, lora_B.T)
    table_wide = table.reshape(table.shape[0] // 2, 2 * EMBED_DIM)
    g_wide = _sc_gather(table_wide, idx >> 1)
    return _tc_select_lora(g_wide, par_t, m_stack, bsz, seq)


# speedup vs baseline: 2.1967x; 1.0414x over previous
import jax
import jax.numpy as jnp
from jax.experimental import pallas as pl
from jax.experimental.pallas import tpu as pltpu
from jax.experimental.pallas import tpu_sc as plsc

EMBED_DIM = 64
RANK_DIM = 8
SCALING = 16.0 / 8.0
GATHER_WINDOW = 256
OUT_BATCH = 128


def _tc_m_stack(a_t, b_t):
    def body(at_ref, bt_ref, m_ref):
        eye = (jax.lax.broadcasted_iota(jnp.int32, (EMBED_DIM, EMBED_DIM), 0)
               == jax.lax.broadcasted_iota(
                   jnp.int32, (EMBED_DIM, EMBED_DIM), 1)).astype(jnp.float32)
        m = eye + SCALING * jnp.dot(at_ref[...], bt_ref[...],
                                    preferred_element_type=jnp.float32)
        m_ref[...] = jnp.concatenate([m, m], axis=0)

    return pl.pallas_call(
        body,
        out_shape=jax.ShapeDtypeStruct((2 * EMBED_DIM, EMBED_DIM),
                                       jnp.float32),
    )(a_t, b_t)


def _sc_gather(table_wide, idx_half):
    n = idx_half.shape[0]
    width = table_wide.shape[1]
    indices = idx_half.reshape(1, n)
    mesh = plsc.VectorSubcoreMesh(core_axis_name="core",
                                  subcore_axis_name="subcore")

    @pl.kernel(out_type=jax.ShapeDtypeStruct((n, width), table_wide.dtype),
               mesh=mesh)
    def gather_kernel(tab_hbm, i_hbm, o_hbm):
        def body(i_vmem, o_vmem):
            pltpu.sync_copy(tab_hbm.at[i_vmem.at[0]], o_vmem)

        pltpu.emit_pipeline(
            body,
            grid=(n // GATHER_WINDOW,),
            in_specs=[pl.BlockSpec((1, GATHER_WINDOW), lambda i: (0, i))],
            out_specs=[pl.BlockSpec((GATHER_WINDOW, width),
                                    lambda i: (i, 0))],
            core_axis_name=("core", "subcore"),
            dimension_semantics=(pltpu.PARALLEL,),
        )(i_hbm, o_hbm)

    return gather_kernel(table_wide, indices)


def _tc_select_lora(g_wide, par_t, m_stack, bsz, seq):
    rows_per_block = OUT_BATCH * seq
    par_cols = rows_per_block // 128

    def body(g_ref, p_ref, m_ref, o_ref):
        gb = g_ref[...]
        parts = []
        for j in range(par_cols):
            lo, hi = j * 128, (j + 1) * 128
            p = p_ref[0, :, j:j + 1]
            mask = jnp.concatenate(
                [jnp.broadcast_to(1.0 - p, (128, EMBED_DIM)),
                 jnp.broadcast_to(p, (128, EMBED_DIM))], axis=1)
            parts.append(gb[lo:hi] * mask)
        sel = jnp.concatenate(parts, axis=0)
        out = jnp.dot(sel, m_ref[...], preferred_element_type=jnp.float32)
        o_ref[...] = out.reshape(OUT_BATCH, seq, EMBED_DIM)

    return pl.pallas_call(
        body,
        grid=(bsz // OUT_BATCH,),
        in_specs=[
            pl.BlockSpec((rows_per_block, 2 * EMBED_DIM), lambda i: (i, 0)),
            pl.BlockSpec((1, 128, par_cols), lambda i: (i, 0, 0)),
            pl.BlockSpec((2 * EMBED_DIM, EMBED_DIM), lambda i: (0, 0)),
        ],
        out_specs=pl.BlockSpec((OUT_BATCH, seq, EMBED_DIM),
                               lambda i: (i, 0, 0)),
        out_shape=jax.ShapeDtypeStruct((bsz, seq, EMBED_DIM), jnp.float32),
    )(g_wide, par_t, m_stack)


def kernel(x, table, lora_A, lora_B):
    bsz, seq = x.shape
    n = bsz * seq
    par_cols = OUT_BATCH * seq // 128
    idx = x.reshape(-1).astype(jnp.int32)
    par_t = ((idx & 1).astype(jnp.float32)
             .reshape(n // 128, 128).T
             .reshape(128, bsz // OUT_BATCH, par_cols)
             .transpose(1, 0, 2))
    m_stack = _tc_m_stack(lora_A.T, lora_B.T)
    table_wide = table.reshape(table.shape[0] // 2, 2 * EMBED_DIM)
    g_wide = _sc_gather(table_wide, idx >> 1)
    return _tc_select_lora(g_wide, par_t, m_stack, bsz, seq)


# speedup vs baseline: 2.1968x; 1.0000x over previous
import jax
import jax.numpy as jnp
from jax.experimental import pallas as pl
from jax.experimental.pallas import tpu as pltpu
from jax.experimental.pallas import tpu_sc as plsc

EMBED_DIM = 64
RANK_DIM = 8
SCALING = 16.0 / 8.0
GATHER_WINDOW = 256
OUT_BATCH = 128
NUM_SLABS = 4


def _tc_m_stack(a_t, b_t):
    def body(at_ref, bt_ref, m_ref):
        eye = (jax.lax.broadcasted_iota(jnp.int32, (EMBED_DIM, EMBED_DIM), 0)
               == jax.lax.broadcasted_iota(
                   jnp.int32, (EMBED_DIM, EMBED_DIM), 1)).astype(jnp.float32)
        m = eye + SCALING * jnp.dot(at_ref[...], bt_ref[...],
                                    preferred_element_type=jnp.float32)
        m_ref[...] = jnp.concatenate([m, m], axis=0)

    return pl.pallas_call(
        body,
        out_shape=jax.ShapeDtypeStruct((2 * EMBED_DIM, EMBED_DIM),
                                       jnp.float32),
    )(a_t, b_t)


def _sc_gather(table_wide, idx_half):
    n = idx_half.shape[0]
    width = table_wide.shape[1]
    indices = idx_half.reshape(1, n)
    mesh = plsc.VectorSubcoreMesh(core_axis_name="core",
                                  subcore_axis_name="subcore")

    @pl.kernel(out_type=jax.ShapeDtypeStruct((n, width), table_wide.dtype),
               mesh=mesh)
    def gather_kernel(tab_hbm, i_hbm, o_hbm):
        def body(i_vmem, o_vmem):
            pltpu.sync_copy(tab_hbm.at[i_vmem.at[0]], o_vmem)

        pltpu.emit_pipeline(
            body,
            grid=(n // GATHER_WINDOW,),
            in_specs=[pl.BlockSpec((1, GATHER_WINDOW), lambda i: (0, i))],
            out_specs=[pl.BlockSpec((GATHER_WINDOW, width),
                                    lambda i: (i, 0))],
            core_axis_name=("core", "subcore"),
            dimension_semantics=(pltpu.PARALLEL,),
        )(i_hbm, o_hbm)

    return gather_kernel(table_wide, indices)


def _tc_select_lora_slab(g_slab, par_slab, m_stack, out_prev, slab, bsz, seq):
    rows_per_block = OUT_BATCH * seq
    par_cols = rows_per_block // 128
    slab_blocks = g_slab.shape[0] // rows_per_block
    block0 = slab * slab_blocks

    def body(g_ref, p_ref, m_ref, *prev_and_out):
        o_ref = prev_and_out[-1]
        gb = g_ref[...]
        parts = []
        for j in range(par_cols):
            lo, hi = j * 128, (j + 1) * 128
            p = p_ref[0, :, j:j + 1]
            mask = jnp.concatenate(
                [jnp.broadcast_to(1.0 - p, (128, EMBED_DIM)),
                 jnp.broadcast_to(p, (128, EMBED_DIM))], axis=1)
            parts.append(gb[lo:hi] * mask)
        sel = jnp.concatenate(parts, axis=0)
        out = jnp.dot(sel, m_ref[...], preferred_element_type=jnp.float32)
        o_ref[...] = out.reshape(OUT_BATCH, seq, EMBED_DIM)

    in_specs = [
        pl.BlockSpec((rows_per_block, 2 * EMBED_DIM), lambda i: (i, 0)),
        pl.BlockSpec((1, 128, par_cols), lambda i: (i, 0, 0)),
        pl.BlockSpec((2 * EMBED_DIM, EMBED_DIM), lambda i: (0, 0)),
    ]
    operands = [g_slab, par_slab, m_stack]
    aliases = {}
    if out_prev is not None:
        in_specs.append(pl.BlockSpec(memory_space=pl.ANY))
        operands.append(out_prev)
        aliases = {3: 0}
    return pl.pallas_call(
        body,
        grid=(slab_blocks,),
        in_specs=in_specs,
        out_specs=pl.BlockSpec((OUT_BATCH, seq, EMBED_DIM),
                               lambda i: (i + block0, 0, 0)),
        out_shape=jax.ShapeDtypeStruct((bsz, seq, EMBED_DIM), jnp.float32),
        input_output_aliases=aliases,
    )(*operands)


def kernel(x, table, lora_A, lora_B):
    bsz, seq = x.shape
    n = bsz * seq
    par_cols = OUT_BATCH * seq // 128
    idx = x.reshape(-1).astype(jnp.int32)
    par_t = ((idx & 1).astype(jnp.float32)
             .reshape(n // 128, 128).T
             .reshape(128, bsz // OUT_BATCH, par_cols)
             .transpose(1, 0, 2))
    m_stack = _tc_m_stack(lora_A.T, lora_B.T)
    table_wide = table.reshape(table.shape[0] // 2, 2 * EMBED_DIM)
    idx_half = idx >> 1

    slab_rows = n // NUM_SLABS
    slab_blocks = bsz // OUT_BATCH // NUM_SLABS
    g_slabs = [
        _sc_gather(table_wide, idx_half[k * slab_rows:(k + 1) * slab_rows])
        for k in range(NUM_SLABS)
    ]
    out = None
    for k in range(NUM_SLABS):
        par_slab = par_t[k * slab_blocks:(k + 1) * slab_blocks]
        out = _tc_select_lora_slab(g_slabs[k], par_slab, m_stack, out, k,
                                   bsz, seq)
    return out
